# Initial kernel scaffold; baseline (speedup 1.0000x reference)
import jax
import jax.numpy as jnp
from jax import lax
from jax.experimental import pallas as pl
from jax.experimental.pallas import tpu as pltpu
from jax.experimental.pallas import tpu_sc as plsc

V = 100000
D = 32
B = 4096
FS = 16
FA = 6
FD = 4
L = 50

NC = 2
NS = 16
NW = NC * NS

PB = B // NW
S_ROWS = PB * FS
A_ROWS = PB * FA
PAIRS = PB * FD
D_ROWS = PAIRS * L

CH = 128
ACC_STRIDE = 520

_mesh = plsc.VectorSubcoreMesh(core_axis_name="c", subcore_axis_name="s")


def _body(stab, sidx, atab, aidx, dtab, didx, dseg, dlens, zeros,
          sout, dout, aout,
          idx_v, seg_v, rows_v, acc_v, lens_v, inv_v, shared, sem):
    c = lax.axis_index("c")
    s = lax.axis_index("s")
    wid = s * NC + c

    for k in range(2):
        pltpu.sync_copy(sidx.at[pl.ds(wid * 16 + k * 8, 8)], idx_v)
        cps = [pltpu.async_copy(stab.at[idx_v.at[j]],
                                rows_v.at[pl.ds(j * CH, CH)], sem)
               for j in range(8)]
        for cp in cps:
            cp.wait()
        pltpu.sync_copy(rows_v, sout.at[pl.ds(wid * S_ROWS + k * 1024, 1024)])

    pltpu.sync_copy(aidx.at[pl.ds(wid * 6, 6)], idx_v.at[pl.ds(0, 6)])
    cps = [pltpu.async_copy(atab.at[idx_v.at[j]],
                            rows_v.at[pl.ds(j * CH, CH)], sem)
           for j in range(6)]
    for cp in cps:
        cp.wait()
    pltpu.sync_copy(rows_v.at[pl.ds(0, A_ROWS)],
                    aout.at[pl.ds(wid * A_ROWS, A_ROWS)])

    pltpu.sync_copy(zeros, shared.at[pl.ds(wid * ACC_STRIDE, PAIRS + 1)])
    for t in range(D_ROWS // 1024):
        pltpu.sync_copy(didx.at[pl.ds(wid * 200 + t * 8, 8)], idx_v)
        pltpu.sync_copy(dseg.at[pl.ds(wid * 200 + t * 8, 8)], seg_v)
        cps = [pltpu.async_copy(dtab.at[idx_v.at[j]],
                                rows_v.at[pl.ds(j * CH, CH)], sem)
               for j in range(8)]
        for cp in cps:
            cp.wait()
        for j in range(8):
            pltpu.sync_copy(rows_v.at[pl.ds(j * CH, CH)],
                            shared.at[seg_v.at[j]], add=True)

    pltpu.sync_copy(dlens.at[pl.ds(wid * PAIRS, PAIRS)], lens_v)
    for i in range(PAIRS // 16):
        lf = lens_v[pl.ds(i * 16, 16)].astype(jnp.float32)
        inv_v[pl.ds(i * 16, 16)] = 1.0 / jnp.maximum(lf, 1.0)

    pltpu.sync_copy(shared.at[pl.ds(wid * ACC_STRIDE, PAIRS)], acc_v)
    ci = lax.iota(jnp.int32, 16)

    @pl.loop(0, PAIRS)
    def _(p):
        rowi = jnp.full((16,), p, dtype=jnp.int32)
        invs = plsc.load_gather(inv_v, [rowi])
        for h in range(2):
            col = ci + h * 16
            v = plsc.load_gather(acc_v, [rowi, col])
            plsc.store_scatter(acc_v, [rowi, col], v * invs)

    pltpu.sync_copy(acc_v, dout.at[pl.ds(wid * PAIRS, PAIRS)])


_emb_call = pl.kernel(
    _body,
    out_type=(
        jax.ShapeDtypeStruct((B * FS, D), jnp.float32),
        jax.ShapeDtypeStruct((B * FD, D), jnp.float32),
        jax.ShapeDtypeStruct((B * FA, D), jnp.float32),
    ),
    mesh=_mesh,
    scratch_types=[
        pltpu.VMEM((8, CH), jnp.int32),
        pltpu.VMEM((8, CH), jnp.int32),
        pltpu.VMEM((1024, D), jnp.float32),
        pltpu.VMEM((PAIRS, D), jnp.float32),
        pltpu.VMEM((PAIRS,), jnp.int32),
        pltpu.VMEM((PAIRS,), jnp.float32),
        pltpu.VMEM_SHARED((NW * ACC_STRIDE, D), jnp.float32),
        pltpu.SemaphoreType.DMA,
    ],
)


def kernel(static_ids, ad_ids, dynamic_ids, dynamic_lengths,
           static_tables, ad_tables, dynamic_tables):
    stab = static_tables.reshape(FS * V, D)
    atab = ad_tables.reshape(FA * V, D)
    dtab = dynamic_tables.reshape(FD * V, D)

    sidx = (static_ids + jnp.arange(FS, dtype=jnp.int32)[None, :] * V
            ).reshape(B * FS // CH, CH)
    aidx = (ad_ids + jnp.arange(FA, dtype=jnp.int32)[None, :] * V
            ).reshape(B * FA // CH, CH)
    didx = (dynamic_ids + (jnp.arange(FD, dtype=jnp.int32) * V)[None, :, None]
            ).reshape(B * FD * L // CH, CH)

    pair = jnp.arange(B * FD, dtype=jnp.int32).reshape(B, FD)
    base = (pair // PAIRS) * ACC_STRIDE
    acc_row = base + pair % PAIRS
    trash = base + PAIRS
    valid = jnp.arange(L, dtype=jnp.int32)[None, None, :] < \
        dynamic_lengths[:, :, None]
    dseg = jnp.where(valid, acc_row[:, :, None], trash[:, :, None]
                     ).astype(jnp.int32).reshape(B * FD * L // CH, CH)

    dlens = dynamic_lengths.reshape(B * FD).astype(jnp.int32)
    zeros = jnp.zeros((PAIRS + 1, D), jnp.float32)

    sout, dout, aout = _emb_call(stab, sidx, atab, aidx, dtab, didx, dseg,
                                 dlens, zeros)

    static_emb = sout.reshape(B, FS, D)
    dyn_emb = dout.reshape(B, FD, D)
    ad_emb = aout.reshape(B, FA, D)
    return (jnp.concatenate([static_emb, dyn_emb], axis=1), ad_emb)


# speedup vs baseline: 3.2232x; 3.2232x over previous
import dataclasses

import jax
import jax.numpy as jnp
from jax import lax
from jax.experimental import pallas as pl
from jax.experimental.pallas import tpu as pltpu
from jax.experimental.pallas import tpu_sc as plsc

V = 100000
D = 32
B = 4096
FS = 16
FA = 6
FD = 4
L = 50

NC = 2
NS = 16
NW = NC * NS

PB = B // NW
S_ROWS = PB * FS
A_ROWS = PB * FA
PAIRS = PB * FD
D_ROWS = PAIRS * L

CH = 128
ACC_STRIDE = 520

_mesh = plsc.VectorSubcoreMesh(core_axis_name="c", subcore_axis_name="s")

_cparams = pltpu.CompilerParams()
if "needs_layout_passes" in pltpu.CompilerParams.__dataclass_fields__:
    _cparams = dataclasses.replace(_cparams, needs_layout_passes=False)
if "use_tc_tiling_on_sc" in pltpu.CompilerParams.__dataclass_fields__:
    _cparams = dataclasses.replace(_cparams, use_tc_tiling_on_sc=False)


def _body(stab, sidx, atab, aidx, dtab, didx, dseg, dlens, zeros,
          sout, dout, aout,
          idx_v, seg_v, rows_v, acc_v, lens_v, inv_v, shared, sem):
    c = lax.axis_index("c")
    s = lax.axis_index("s")
    wid = s * NC + c

    for k in range(2):
        pltpu.sync_copy(sidx.at[pl.ds(wid * 16 + k * 8, 8)], idx_v)
        cps = [pltpu.async_copy(stab.at[idx_v.at[j]],
                                rows_v.at[pl.ds(j * CH, CH)], sem)
               for j in range(8)]
        for cp in cps:
            cp.wait()
        pltpu.sync_copy(rows_v, sout.at[pl.ds(wid * S_ROWS + k * 1024, 1024)])

    pltpu.sync_copy(aidx.at[pl.ds(wid * 8, 8)], idx_v)
    cps = [pltpu.async_copy(atab.at[idx_v.at[j]],
                            rows_v.at[pl.ds(j * CH, CH)], sem)
           for j in range(8)]
    for cp in cps:
        cp.wait()
    pltpu.sync_copy(rows_v.at[pl.ds(0, A_ROWS)],
                    aout.at[pl.ds(wid * A_ROWS, A_ROWS)])

    pltpu.sync_copy(zeros, shared.at[pl.ds(wid * ACC_STRIDE, ACC_STRIDE)])
    for t in range(D_ROWS // 1024):
        pltpu.sync_copy(didx.at[pl.ds(wid * 200 + t * 8, 8)], idx_v)
        pltpu.sync_copy(dseg.at[pl.ds(wid * 200 + t * 8, 8)], seg_v)
        cps = [pltpu.async_copy(dtab.at[idx_v.at[j]],
                                rows_v.at[pl.ds(j * CH, CH)], sem)
               for j in range(8)]
        for cp in cps:
            cp.wait()
        for j in range(8):
            pltpu.sync_copy(rows_v.at[pl.ds(j * CH, CH)],
                            shared.at[seg_v.at[j]], add=True)

    pltpu.sync_copy(dlens.at[pl.ds(wid * PAIRS, PAIRS)], lens_v)
    for i in range(PAIRS // 16):
        lf = lens_v[pl.ds(i * 16, 16)].astype(jnp.float32)
        inv_v[pl.ds(i * 16, 16)] = 1.0 / jnp.maximum(lf, 1.0)

    pltpu.sync_copy(shared.at[pl.ds(wid * ACC_STRIDE, PAIRS)], acc_v)
    ci = lax.iota(jnp.int32, 16)

    @pl.loop(0, PAIRS)
    def _(p):
        rowi = jnp.full((16,), p, dtype=jnp.int32)
        invs = plsc.load_gather(inv_v, [rowi])
        for h in range(2):
            col = ci + h * 16
            v = plsc.load_gather(acc_v, [rowi, col])
            plsc.store_scatter(acc_v, [rowi, col], v * invs)

    pltpu.sync_copy(acc_v, dout.at[pl.ds(wid * PAIRS, PAIRS)])


_emb_call = pl.kernel(
    _body,
    out_type=(
        jax.ShapeDtypeStruct((B * FS, D), jnp.float32),
        jax.ShapeDtypeStruct((B * FD, D), jnp.float32),
        jax.ShapeDtypeStruct((B * FA, D), jnp.float32),
    ),
    mesh=_mesh,
    scratch_types=[
        pltpu.VMEM((8, CH), jnp.int32),
        pltpu.VMEM((8, CH), jnp.int32),
        pltpu.VMEM((1024, D), jnp.float32),
        pltpu.VMEM((PAIRS, D), jnp.float32),
        pltpu.VMEM((PAIRS,), jnp.int32),
        pltpu.VMEM((PAIRS,), jnp.float32),
        pltpu.VMEM_SHARED((NW * ACC_STRIDE, D), jnp.float32),
        pltpu.SemaphoreType.DMA,
    ],
    compiler_params=_cparams,
)


def kernel(static_ids, ad_ids, dynamic_ids, dynamic_lengths,
           static_tables, ad_tables, dynamic_tables):
    stab = static_tables.reshape(FS * V, D)
    atab = ad_tables.reshape(FA * V, D)
    dtab = dynamic_tables.reshape(FD * V, D)

    sidx = (static_ids + jnp.arange(FS, dtype=jnp.int32)[None, :] * V
            ).reshape(B * FS // CH, CH)
    aidx = (ad_ids + jnp.arange(FA, dtype=jnp.int32)[None, :] * V
            ).reshape(NW, A_ROWS)
    aidx = jnp.concatenate(
        [aidx, jnp.zeros((NW, 8 * CH - A_ROWS), jnp.int32)], axis=1
    ).reshape(NW * 8, CH)
    didx = (dynamic_ids + (jnp.arange(FD, dtype=jnp.int32) * V)[None, :, None]
            ).reshape(B * FD * L // CH, CH)

    pair = jnp.arange(B * FD, dtype=jnp.int32).reshape(B, FD)
    base = (pair // PAIRS) * ACC_STRIDE
    acc_row = base + pair % PAIRS
    trash = base + PAIRS
    valid = jnp.arange(L, dtype=jnp.int32)[None, None, :] < \
        dynamic_lengths[:, :, None]
    dseg = jnp.where(valid, acc_row[:, :, None], trash[:, :, None]
                     ).astype(jnp.int32).reshape(B * FD * L // CH, CH)

    dlens = dynamic_lengths.reshape(B * FD).astype(jnp.int32)
    zeros = jnp.zeros((ACC_STRIDE, D), jnp.float32)

    sout, dout, aout = _emb_call(stab, sidx, atab, aidx, dtab, didx, dseg,
                                 dlens, zeros)

    static_emb = sout.reshape(B, FS, D)
    dyn_emb = dout.reshape(B, FD, D)
    ad_emb = aout.reshape(B, FA, D)
    return (jnp.concatenate([static_emb, dyn_emb], axis=1), ad_emb)


# speedup vs baseline: 3.4780x; 1.0791x over previous
import dataclasses

import jax
import jax.numpy as jnp
from jax import lax
from jax.experimental import pallas as pl
from jax.experimental.pallas import tpu as pltpu
from jax.experimental.pallas import tpu_sc as plsc

V = 100000
D = 32
B = 4096
FS = 16
FA = 6
FD = 4
L = 50
FC = FS + FD

NC = 2
NS = 16
NW = NC * NS

PB = B // NW
S_ROWS = PB * FS
A_ROWS = PB * FA
PAIRS = PB * FD
D_ROWS = PAIRS * L

CH = 128
ACC_STRIDE = 520

_mesh = plsc.VectorSubcoreMesh(core_axis_name="c", subcore_axis_name="s")

_cparams = pltpu.CompilerParams()
if "needs_layout_passes" in pltpu.CompilerParams.__dataclass_fields__:
    _cparams = dataclasses.replace(_cparams, needs_layout_passes=False)
if "use_tc_tiling_on_sc" in pltpu.CompilerParams.__dataclass_fields__:
    _cparams = dataclasses.replace(_cparams, use_tc_tiling_on_sc=False)


def _body(stab, atab, dtab, sids, aids, dids, dlens, zeros,
          out1, aout,
          idx_v, seg_v, dst_v, dstd_v, rows_v, acc_v, lens_v, inv_v,
          shared, sem):
    c = lax.axis_index("c")
    s = lax.axis_index("s")
    wid = s * NC + c

    ci = lax.iota(jnp.int32, 16)

    def vec16(ref, off16):
        return plsc.load_gather(ref, [off16])

    ioff = ci * V
    for k in range(2):
        base = wid * S_ROWS + k * 1024
        pltpu.sync_copy(sids.at[pl.ds(base, 1024)], idx_v)

        @pl.loop(0, 64)
        def _(m):
            off16 = m * 16 + ci
            idxv = vec16(idx_v, off16) + ioff
            plsc.store_scatter(idx_v, [off16], idxv)
            posv = off16 + base
            dest = (posv >> 4) * FC + (posv & 15)
            plsc.store_scatter(dst_v, [jnp.full((16,), m // 8, jnp.int32),
                                       (m % 8) * 16 + ci], dest)

        cps = [pltpu.async_copy(stab.at[idx_v.at[pl.ds(j * CH, CH)]],
                                rows_v.at[pl.ds(j * CH, CH)], sem)
               for j in range(8)]
        for cp in cps:
            cp.wait()
        for j in range(8):
            pltpu.sync_copy(rows_v.at[pl.ds(j * CH, CH)],
                            out1.at[dst_v.at[j]])

    abase = wid * A_ROWS
    pltpu.sync_copy(aids.at[pl.ds(abase, A_ROWS)], idx_v.at[pl.ds(0, A_ROWS)])

    @pl.loop(0, 48)
    def _(m):
        off16 = m * 16 + ci
        posv = off16 + abase
        f = posv % 6
        idxv = vec16(idx_v, off16) + f * V
        plsc.store_scatter(idx_v, [off16], idxv)

    cps = [pltpu.async_copy(atab.at[idx_v.at[pl.ds(j * CH, CH)]],
                            rows_v.at[pl.ds(j * CH, CH)], sem)
           for j in range(6)]
    for cp in cps:
        cp.wait()
    pltpu.sync_copy(rows_v.at[pl.ds(0, A_ROWS)],
                    aout.at[pl.ds(abase, A_ROWS)])

    pltpu.sync_copy(dlens.at[pl.ds(wid * PAIRS, PAIRS)], lens_v)
    pltpu.sync_copy(zeros, shared.at[pl.ds(wid * ACC_STRIDE, ACC_STRIDE)])
    accbase = wid * ACC_STRIDE
    for t in range(D_ROWS // 1024):
        cbase = wid * D_ROWS + t * 1024
        pltpu.sync_copy(dids.at[pl.ds(cbase, 1024)], idx_v)

        @pl.loop(0, 64)
        def _(m):
            off16 = m * 16 + ci
            posv = off16 + cbase
            pair = posv // L
            idxv = vec16(idx_v, off16) + (pair & (FD - 1)) * V
            plsc.store_scatter(idx_v, [off16], idxv)
            local = pair & (PAIRS - 1)
            ln = posv - pair * L
            lenv = vec16(lens_v, local)
            segv = jnp.where(ln < lenv, local, PAIRS) + accbase
            plsc.store_scatter(seg_v, [jnp.full((16,), m // 8, jnp.int32),
                                       (m % 8) * 16 + ci], segv)

        cps = [pltpu.async_copy(dtab.at[idx_v.at[pl.ds(j * CH, CH)]],
                                rows_v.at[pl.ds(j * CH, CH)], sem)
               for j in range(8)]
        for cp in cps:
            cp.wait()
        for j in range(8):
            pltpu.sync_copy(rows_v.at[pl.ds(j * CH, CH)],
                            shared.at[seg_v.at[j]], add=True)

    for i in range(PAIRS // 16):
        lf = lens_v[pl.ds(i * 16, 16)].astype(jnp.float32)
        inv_v[pl.ds(i * 16, 16)] = 1.0 / jnp.maximum(lf, 1.0)

    pltpu.sync_copy(shared.at[pl.ds(accbase, PAIRS)], acc_v)

    @pl.loop(0, PAIRS)
    def _(p):
        rowi = jnp.full((16,), p, dtype=jnp.int32)
        invs = plsc.load_gather(inv_v, [rowi])
        for h in range(2):
            col = ci + h * 16
            v = plsc.load_gather(acc_v, [rowi, col])
            plsc.store_scatter(acc_v, [rowi, col], v * invs)

    @pl.loop(0, PAIRS // 16)
    def _(m):
        pv = m * 16 + ci
        dest = (wid * PB + (pv >> 2)) * FC + FS + (pv & 3)
        plsc.store_scatter(dstd_v, [jnp.full((16,), m // 8, jnp.int32),
                                    (m % 8) * 16 + ci], dest)

    for j in range(PAIRS // CH):
        pltpu.sync_copy(acc_v.at[pl.ds(j * CH, CH)], out1.at[dstd_v.at[j]])


_emb_call = pl.kernel(
    _body,
    out_type=(
        jax.ShapeDtypeStruct((B * FC, D), jnp.float32),
        jax.ShapeDtypeStruct((B * FA, D), jnp.float32),
    ),
    mesh=_mesh,
    scratch_types=[
        pltpu.VMEM((1024,), jnp.int32),
        pltpu.VMEM((8, CH), jnp.int32),
        pltpu.VMEM((8, CH), jnp.int32),
        pltpu.VMEM((4, CH), jnp.int32),
        pltpu.VMEM((1024, D), jnp.float32),
        pltpu.VMEM((PAIRS, D), jnp.float32),
        pltpu.VMEM((PAIRS,), jnp.int32),
        pltpu.VMEM((PAIRS,), jnp.float32),
        pltpu.VMEM_SHARED((NW * ACC_STRIDE, D), jnp.float32),
        pltpu.SemaphoreType.DMA,
    ],
    compiler_params=_cparams,
)


def kernel(static_ids, ad_ids, dynamic_ids, dynamic_lengths,
           static_tables, ad_tables, dynamic_tables):
    stab = static_tables.reshape(FS * V, D)
    atab = ad_tables.reshape(FA * V, D)
    dtab = dynamic_tables.reshape(FD * V, D)

    sids = static_ids.reshape(B * FS)
    aids = ad_ids.reshape(B * FA)
    dids = dynamic_ids.reshape(B * FD * L)
    dlens = dynamic_lengths.reshape(B * FD)
    zeros = jnp.zeros((ACC_STRIDE, D), jnp.float32)

    out1, aout = _emb_call(stab, atab, dtab, sids, aids, dids, dlens, zeros)
    return (out1.reshape(B, FC, D), aout.reshape(B, FA, D))


# speedup vs baseline: 3.4953x; 1.0050x over previous
import dataclasses

import jax
import jax.numpy as jnp
from jax import lax
from jax.experimental import pallas as pl
from jax.experimental.pallas import tpu as pltpu
from jax.experimental.pallas import tpu_sc as plsc

V = 100000
D = 32
B = 4096
FS = 16
FA = 6
FD = 4
L = 50
FC = FS + FD

NC = 2
NS = 16
NW = NC * NS

PB = B // NW
S_ROWS = PB * FS
A_ROWS = PB * FA
PAIRS = PB * FD
D_ROWS = PAIRS * L

CH = 128
ACC_STRIDE = 520

_mesh = plsc.VectorSubcoreMesh(core_axis_name="c", subcore_axis_name="s")

_cparams = pltpu.CompilerParams()
if "needs_layout_passes" in pltpu.CompilerParams.__dataclass_fields__:
    _cparams = dataclasses.replace(_cparams, needs_layout_passes=False)
if "use_tc_tiling_on_sc" in pltpu.CompilerParams.__dataclass_fields__:
    _cparams = dataclasses.replace(_cparams, use_tc_tiling_on_sc=False)


def _body(stab, atab, dtab, sids, aids, dids, dlens, zeros,
          out1, aout,
          idx_v, seg_v, rows_v, sids_v, aids_v, dids_v, lens_v, inv_v,
          shared, sem):
    c = lax.axis_index("c")
    s = lax.axis_index("s")
    wid = s * NC + c
    b0 = wid * PB

    ci = lax.iota(jnp.int32, 16)

    pltpu.sync_copy(sids.at[pl.ds(b0, PB)], sids_v)
    pltpu.sync_copy(aids.at[pl.ds(b0, PB)], aids_v)
    pltpu.sync_copy(dids.at[pl.ds(b0, PB)], dids_v)
    pltpu.sync_copy(dlens.at[pl.ds(b0, PB)], lens_v)

    for k in range(2):
        @pl.loop(0, 64)
        def _(m):
            pv = k * 1024 + m * 16 + ci
            bl = pv >> 4
            f = pv & 15
            idxv = plsc.load_gather(sids_v, [bl, f]) + f * V
            plsc.store_scatter(idx_v, [m * 16 + ci], idxv)

        cps = [pltpu.async_copy(stab.at[idx_v.at[pl.ds(j * CH, CH)]],
                                rows_v.at[pl.ds(j * CH, CH)], sem)
               for j in range(8)]
        for cp in cps:
            cp.wait()
        cps = [pltpu.async_copy(rows_v.at[pl.ds(bi * FS, FS)],
                                out1.at[b0 + k * 64 + bi].at[pl.ds(0, FS)],
                                sem)
               for bi in range(64)]
        for cp in cps:
            cp.wait()

    @pl.loop(0, 48)
    def _(m):
        pv = m * 16 + ci
        bl = pv // FA
        f = pv - bl * FA
        idxv = plsc.load_gather(aids_v, [bl, f]) + f * V
        plsc.store_scatter(idx_v, [m * 16 + ci], idxv)

    cps = [pltpu.async_copy(atab.at[idx_v.at[pl.ds(j * CH, CH)]],
                            rows_v.at[pl.ds(j * CH, CH)], sem)
           for j in range(6)]
    for cp in cps:
        cp.wait()
    cps = [pltpu.async_copy(rows_v.at[pl.ds(bi * FA, FA)],
                            aout.at[b0 + bi], sem)
           for bi in range(PB)]
    for cp in cps:
        cp.wait()

    pltpu.sync_copy(zeros, shared.at[pl.ds(wid * ACC_STRIDE, ACC_STRIDE)])
    accbase = wid * ACC_STRIDE
    for t in range(D_ROWS // 1024):
        @pl.loop(0, 64)
        def _(m):
            q = t * 1024 + m * 16 + ci
            pairl = q // L
            f = pairl & (FD - 1)
            bl = pairl >> 2
            ln = q - pairl * L
            idxv = plsc.load_gather(dids_v, [bl, f, ln]) + f * V
            plsc.store_scatter(idx_v, [m * 16 + ci], idxv)
            lenv = plsc.load_gather(lens_v, [bl, f])
            segv = jnp.where(ln < lenv, pairl, PAIRS) + accbase
            plsc.store_scatter(seg_v, [jnp.full((16,), m // 8, jnp.int32),
                                       (m % 8) * 16 + ci], segv)

        cps = [pltpu.async_copy(dtab.at[idx_v.at[pl.ds(j * CH, CH)]],
                                rows_v.at[pl.ds(j * CH, CH)], sem)
               for j in range(8)]
        for cp in cps:
            cp.wait()
        for j in range(8):
            pltpu.sync_copy(rows_v.at[pl.ds(j * CH, CH)],
                            shared.at[seg_v.at[j]], add=True)

    for i in range(PAIRS // 16):
        pv = i * 16 + ci
        lf = plsc.load_gather(lens_v, [pv >> 2, pv & 3]).astype(jnp.float32)
        inv_v[pl.ds(i * 16, 16)] = 1.0 / jnp.maximum(lf, 1.0)

    pltpu.sync_copy(shared.at[pl.ds(accbase, PAIRS)], rows_v.at[pl.ds(0, PAIRS)])

    @pl.loop(0, PAIRS)
    def _(p):
        rowi = jnp.full((16,), p, dtype=jnp.int32)
        invs = plsc.load_gather(inv_v, [rowi])
        for h in range(2):
            col = ci + h * 16
            v = plsc.load_gather(rows_v, [rowi, col])
            plsc.store_scatter(rows_v, [rowi, col], v * invs)

    cps = [pltpu.async_copy(rows_v.at[pl.ds(bi * FD, FD)],
                            out1.at[b0 + bi].at[pl.ds(FS, FD)], sem)
           for bi in range(PB)]
    for cp in cps:
        cp.wait()


_emb_call = pl.kernel(
    _body,
    out_type=(
        jax.ShapeDtypeStruct((B, FC, D), jnp.float32),
        jax.ShapeDtypeStruct((B, FA, D), jnp.float32),
    ),
    mesh=_mesh,
    scratch_types=[
        pltpu.VMEM((1024,), jnp.int32),
        pltpu.VMEM((8, CH), jnp.int32),
        pltpu.VMEM((1024, D), jnp.float32),
        pltpu.VMEM((PB, FS), jnp.int32),
        pltpu.VMEM((PB, FA), jnp.int32),
        pltpu.VMEM((PB, FD, L), jnp.int32),
        pltpu.VMEM((PB, FD), jnp.int32),
        pltpu.VMEM((PAIRS,), jnp.float32),
        pltpu.VMEM_SHARED((NW * ACC_STRIDE, D), jnp.float32),
        pltpu.SemaphoreType.DMA,
    ],
    compiler_params=_cparams,
)


def kernel(static_ids, ad_ids, dynamic_ids, dynamic_lengths,
           static_tables, ad_tables, dynamic_tables):
    stab = static_tables.reshape(FS * V, D)
    atab = ad_tables.reshape(FA * V, D)
    dtab = dynamic_tables.reshape(FD * V, D)
    zeros = jnp.zeros((ACC_STRIDE, D), jnp.float32)
    out1, aout = _emb_call(stab, atab, dtab, static_ids, ad_ids,
                           dynamic_ids, dynamic_lengths, zeros)
    return (out1, aout)


# speedup vs baseline: 3.5941x; 1.0283x over previous
import dataclasses

import jax
import jax.numpy as jnp
from jax import lax
from jax.experimental import pallas as pl
from jax.experimental.pallas import tpu as pltpu
from jax.experimental.pallas import tpu_sc as plsc

V = 100000
D = 32
B = 4096
FS = 16
FA = 6
FD = 4
L = 50
FC = FS + FD

NC = 2
NS = 16
NW = NC * NS

PB = B // NW
A_ROWS = PB * FA
PAIRS = PB * FD
DF_ROWS = PB * L

CH = 128
DCH = 640
ACC_STRIDE = 520

_mesh = plsc.VectorSubcoreMesh(core_axis_name="c", subcore_axis_name="s")

_cparams = pltpu.CompilerParams()
if "needs_layout_passes" in pltpu.CompilerParams.__dataclass_fields__:
    _cparams = dataclasses.replace(_cparams, needs_layout_passes=False)
if "use_tc_tiling_on_sc" in pltpu.CompilerParams.__dataclass_fields__:
    _cparams = dataclasses.replace(_cparams, use_tc_tiling_on_sc=False)


def _body(stab, atab, dtab, sids, aids, dids, dlens, zeros,
          out1, aout,
          idx_v, seg_v, dst_v, rows_v, sids_v, aids_v, dids_v, lens_v, inv_v,
          shared, sem):
    c = lax.axis_index("c")
    s = lax.axis_index("s")
    wid = s * NC + c
    b0 = wid * PB

    ci = lax.iota(jnp.int32, 16)

    pltpu.sync_copy(sids.at[pl.ds(b0, PB)], sids_v)
    pltpu.sync_copy(aids.at[pl.ds(b0, PB)], aids_v)
    pltpu.sync_copy(dids.at[pl.ds(b0, PB)], dids_v)
    pltpu.sync_copy(dlens.at[pl.ds(b0, PB)], lens_v)

    @pl.loop(0, 128)
    def _(m):
        pv = m * 16 + ci
        f = pv >> 7
        bl = pv & (PB - 1)
        plsc.store_scatter(idx_v, [pv], plsc.load_gather(sids_v, [bl, f]))
        plsc.store_scatter(dst_v, [jnp.full((16,), m // 8, jnp.int32),
                                   (m % 8) * 16 + ci],
                           (b0 + bl) * FC + f)

    cps = [pltpu.async_copy(stab.at[f].at[idx_v.at[pl.ds(f * CH, CH)]],
                            rows_v.at[pl.ds(f * CH, CH)], sem)
           for f in range(FS)]
    for cp in cps:
        cp.wait()
    for f in range(FS):
        pltpu.sync_copy(rows_v.at[pl.ds(f * CH, CH)], out1.at[dst_v.at[f]])

    @pl.loop(0, 48)
    def _(m):
        pv = m * 16 + ci
        f = pv >> 7
        bl = pv & (PB - 1)
        plsc.store_scatter(idx_v, [pv], plsc.load_gather(aids_v, [bl, f]))
        plsc.store_scatter(dst_v, [jnp.full((16,), m // 8, jnp.int32),
                                   (m % 8) * 16 + ci],
                           (b0 + bl) * FA + f)

    cps = [pltpu.async_copy(atab.at[f].at[idx_v.at[pl.ds(f * CH, CH)]],
                            rows_v.at[pl.ds(f * CH, CH)], sem)
           for f in range(FA)]
    for cp in cps:
        cp.wait()
    for f in range(FA):
        pltpu.sync_copy(rows_v.at[pl.ds(f * CH, CH)], aout.at[dst_v.at[f]])

    pltpu.sync_copy(zeros, shared.at[pl.ds(s * ACC_STRIDE, ACC_STRIDE)])
    accbase = s * ACC_STRIDE
    for f in range(FD):
        for t in range(DF_ROWS // DCH):
            @pl.loop(0, DCH // 16)
            def _(m):
                qf = t * DCH + m * 16 + ci
                bl = qf // L
                ln = qf - bl * L
                pairl = bl * FD + f
                idxv = plsc.load_gather(dids_v, [bl, jnp.full((16,), f,
                                                              jnp.int32), ln])
                plsc.store_scatter(idx_v, [m * 16 + ci], idxv)
                lenv = plsc.load_gather(lens_v, [bl, jnp.full((16,), f,
                                                              jnp.int32)])
                segv = jnp.where(ln < lenv, pairl, PAIRS) + accbase
                plsc.store_scatter(seg_v, [jnp.full((16,), m // 8, jnp.int32),
                                           (m % 8) * 16 + ci], segv)

            cps = [pltpu.async_copy(dtab.at[f].at[idx_v.at[pl.ds(j * CH, CH)]],
                                    rows_v.at[pl.ds(j * CH, CH)], sem)
                   for j in range(DCH // CH)]
            for cp in cps:
                cp.wait()
            for j in range(DCH // CH):
                pltpu.sync_copy(rows_v.at[pl.ds(j * CH, CH)],
                                shared.at[seg_v.at[j]], add=True)

    for i in range(PAIRS // 16):
        pv = i * 16 + ci
        lf = plsc.load_gather(lens_v, [pv >> 2, pv & 3]).astype(jnp.float32)
        inv_v[pl.ds(i * 16, 16)] = 1.0 / jnp.maximum(lf, 1.0)

    pltpu.sync_copy(shared.at[pl.ds(accbase, PAIRS)],
                    rows_v.at[pl.ds(0, PAIRS)])

    @pl.loop(0, PAIRS)
    def _(p):
        rowi = jnp.full((16,), p, dtype=jnp.int32)
        invs = plsc.load_gather(inv_v, [rowi])
        for h in range(2):
            col = ci + h * 16
            v = plsc.load_gather(rows_v, [rowi, col])
            plsc.store_scatter(rows_v, [rowi, col], v * invs)

    @pl.loop(0, PAIRS // 16)
    def _(m):
        pv = m * 16 + ci
        dest = (b0 + (pv >> 2)) * FC + FS + (pv & 3)
        plsc.store_scatter(dst_v, [jnp.full((16,), m // 8, jnp.int32),
                                   (m % 8) * 16 + ci], dest)

    for j in range(PAIRS // CH):
        pltpu.sync_copy(rows_v.at[pl.ds(j * CH, CH)], out1.at[dst_v.at[j]])


_emb_call = pl.kernel(
    _body,
    out_type=(
        jax.ShapeDtypeStruct((B * FC, D), jnp.float32),
        jax.ShapeDtypeStruct((B * FA, D), jnp.float32),
    ),
    mesh=_mesh,
    scratch_types=[
        pltpu.VMEM((2048,), jnp.int32),
        pltpu.VMEM((8, CH), jnp.int32),
        pltpu.VMEM((16, CH), jnp.int32),
        pltpu.VMEM((2048, D), jnp.float32),
        pltpu.VMEM((PB, FS), jnp.int32),
        pltpu.VMEM((PB, FA), jnp.int32),
        pltpu.VMEM((PB, FD, L), jnp.int32),
        pltpu.VMEM((PB, FD), jnp.int32),
        pltpu.VMEM((PAIRS,), jnp.float32),
        pltpu.VMEM_SHARED((NS * ACC_STRIDE, D), jnp.float32),
        pltpu.SemaphoreType.DMA,
    ],
    compiler_params=_cparams,
)


def kernel(static_ids, ad_ids, dynamic_ids, dynamic_lengths,
           static_tables, ad_tables, dynamic_tables):
    zeros = jnp.zeros((ACC_STRIDE, D), jnp.float32)
    out1, aout = _emb_call(static_tables, ad_tables, dynamic_tables,
                           static_ids, ad_ids, dynamic_ids, dynamic_lengths,
                           zeros)
    return (out1.reshape(B, FC, D), aout.reshape(B, FA, D))


# speedup vs baseline: 3.7865x; 1.0535x over previous
import dataclasses
import functools

import jax
import jax.numpy as jnp
from jax import lax
from jax.experimental import pallas as pl
from jax.experimental.pallas import tpu as pltpu
from jax.experimental.pallas import tpu_sc as plsc

V = 100000
D = 32
B = 4096
FS = 16
FA = 6
FD = 4
L = 50
FC = FS + FD

NC = 2
NS = 16
NW = NC * NS

PB = B // NW
PAIRS = PB * FD
DF_ROWS = PB * L

CH = 128
DCH = 640
ACC_STRIDE = 520

_mesh = plsc.VectorSubcoreMesh(core_axis_name="c", subcore_axis_name="s")

_cparams = pltpu.CompilerParams()
if "needs_layout_passes" in pltpu.CompilerParams.__dataclass_fields__:
    _cparams = dataclasses.replace(_cparams, needs_layout_passes=False)
if "use_tc_tiling_on_sc" in pltpu.CompilerParams.__dataclass_fields__:
    _cparams = dataclasses.replace(_cparams, use_tc_tiling_on_sc=False)

_ci = functools.partial(lax.iota, jnp.int32)


def _lookup_body(nf, tab, ids, out, idx_v, dst_v, rows_v, ids_v, sem):
    c = lax.axis_index("c")
    s = lax.axis_index("s")
    b0 = (s * NC + c) * PB
    ci = _ci(16)

    pltpu.sync_copy(ids.at[pl.ds(b0, PB)], ids_v)

    @pl.loop(0, nf * 8)
    def _(m):
        pv = m * 16 + ci
        f = pv >> 7
        bl = pv & (PB - 1)
        plsc.store_scatter(idx_v, [pv], plsc.load_gather(ids_v, [bl, f]))
        plsc.store_scatter(dst_v, [jnp.full((16,), m // 8, jnp.int32),
                                   (m % 8) * 16 + ci],
                           (b0 + bl) * nf + f)

    cps = [pltpu.async_copy(tab.at[f].at[idx_v.at[pl.ds(f * CH, CH)]],
                            rows_v.at[pl.ds(f * CH, CH)], sem)
           for f in range(nf)]
    for cp in cps:
        cp.wait()
    for f in range(nf):
        pltpu.sync_copy(rows_v.at[pl.ds(f * CH, CH)], out.at[dst_v.at[f]])


def _dyn_body(dtab, dids, dlens, zeros, out,
              idx_v, seg_v, dst_v, rows_v, dids_v, lens_v, inv_v,
              shared, sem):
    c = lax.axis_index("c")
    s = lax.axis_index("s")
    wid = s * NC + c
    b0 = wid * PB
    ci = _ci(16)

    pltpu.sync_copy(dids.at[pl.ds(b0, PB)], dids_v)
    pltpu.sync_copy(dlens.at[pl.ds(b0, PB)], lens_v)
    pltpu.sync_copy(zeros, shared.at[pl.ds(s * ACC_STRIDE, ACC_STRIDE)])
    accbase = s * ACC_STRIDE

    for f in range(FD):
        for t in range(DF_ROWS // DCH):
            @pl.loop(0, DCH // 16)
            def _(m):
                qf = t * DCH + m * 16 + ci
                bl = qf // L
                ln = qf - bl * L
                pairl = bl * FD + f
                fv = jnp.full((16,), f, jnp.int32)
                idxv = plsc.load_gather(dids_v, [bl, fv, ln])
                plsc.store_scatter(idx_v, [m * 16 + ci], idxv)
                lenv = plsc.load_gather(lens_v, [bl, fv])
                segv = jnp.where(ln < lenv, pairl, PAIRS) + accbase
                plsc.store_scatter(seg_v, [jnp.full((16,), m // 8, jnp.int32),
                                           (m % 8) * 16 + ci], segv)

            cps = [pltpu.async_copy(dtab.at[f].at[idx_v.at[pl.ds(j * CH, CH)]],
                                    rows_v.at[pl.ds(j * CH, CH)], sem)
                   for j in range(DCH // CH)]
            for cp in cps:
                cp.wait()
            for j in range(DCH // CH):
                pltpu.sync_copy(rows_v.at[pl.ds(j * CH, CH)],
                                shared.at[seg_v.at[j]], add=True)

    for i in range(PAIRS // 16):
        pv = i * 16 + ci
        lf = plsc.load_gather(lens_v, [pv >> 2, pv & 3]).astype(jnp.float32)
        inv_v[pl.ds(i * 16, 16)] = 1.0 / jnp.maximum(lf, 1.0)

    pltpu.sync_copy(shared.at[pl.ds(accbase, PAIRS)],
                    rows_v.at[pl.ds(0, PAIRS)])

    @pl.loop(0, PAIRS)
    def _(p):
        rowi = jnp.full((16,), p, dtype=jnp.int32)
        invs = plsc.load_gather(inv_v, [rowi])
        for h in range(2):
            col = ci + h * 16
            v = plsc.load_gather(rows_v, [rowi, col])
            plsc.store_scatter(rows_v, [rowi, col], v * invs)

    pltpu.sync_copy(rows_v.at[pl.ds(0, PAIRS)],
                    out.at[pl.ds(b0 * FD, PAIRS)])


def _mk_lookup(nf):
    return pl.kernel(
        functools.partial(_lookup_body, nf),
        out_type=jax.ShapeDtypeStruct((B * nf, D), jnp.float32),
        mesh=_mesh,
        scratch_types=[
            pltpu.VMEM((nf * CH,), jnp.int32),
            pltpu.VMEM((nf, CH), jnp.int32),
            pltpu.VMEM((nf * CH, D), jnp.float32),
            pltpu.VMEM((PB, nf), jnp.int32),
            pltpu.SemaphoreType.DMA,
        ],
        compiler_params=_cparams,
    )


_static_call = _mk_lookup(FS)
_ad_call = _mk_lookup(FA)

_dyn_call = pl.kernel(
    _dyn_body,
    out_type=jax.ShapeDtypeStruct((B * FD, D), jnp.float32),
    mesh=_mesh,
    scratch_types=[
        pltpu.VMEM((DCH,), jnp.int32),
        pltpu.VMEM((8, CH), jnp.int32),
        pltpu.VMEM((4, CH), jnp.int32),
        pltpu.VMEM((DCH, D), jnp.float32),
        pltpu.VMEM((PB, FD, L), jnp.int32),
        pltpu.VMEM((PB, FD), jnp.int32),
        pltpu.VMEM((PAIRS,), jnp.float32),
        pltpu.VMEM_SHARED((NS * ACC_STRIDE, D), jnp.float32),
        pltpu.SemaphoreType.DMA,
    ],
    compiler_params=_cparams,
)


def kernel(static_ids, ad_ids, dynamic_ids, dynamic_lengths,
           static_tables, ad_tables, dynamic_tables):
    zeros = jnp.zeros((ACC_STRIDE, D), jnp.float32)
    dout = _dyn_call(dynamic_tables, dynamic_ids, dynamic_lengths, zeros)
    sout = _static_call(static_tables, static_ids)
    aout = _ad_call(ad_tables, ad_ids)
    out1 = jnp.concatenate(
        [sout.reshape(B, FS, D), dout.reshape(B, FD, D)], axis=1)
    return (out1, aout.reshape(B, FA, D))


# speedup vs baseline: 3.7954x; 1.0023x over previous
import dataclasses
import functools

import jax
import jax.numpy as jnp
from jax import lax
from jax.experimental import pallas as pl
from jax.experimental.pallas import tpu as pltpu
from jax.experimental.pallas import tpu_sc as plsc

V = 100000
D = 32
B = 4096
FS = 16
FA = 6
FD = 4
L = 50
FC = FS + FD

NC = 2
NS = 16
NW = NC * NS

PB = B // NW
PAIRS = PB * FD
DF_ROWS = PB * L

CH = 128
DCH = 640
ACC_STRIDE = 520

_mesh = plsc.VectorSubcoreMesh(core_axis_name="c", subcore_axis_name="s")

_cparams = pltpu.CompilerParams()
if "needs_layout_passes" in pltpu.CompilerParams.__dataclass_fields__:
    _cparams = dataclasses.replace(_cparams, needs_layout_passes=False)
if "use_tc_tiling_on_sc" in pltpu.CompilerParams.__dataclass_fields__:
    _cparams = dataclasses.replace(_cparams, use_tc_tiling_on_sc=False)

_ci = functools.partial(lax.iota, jnp.int32)


def _lookup_body(nf, tab, ids, out, idx_v, dst_v, rows_v, ids_v, sem):
    c = lax.axis_index("c")
    s = lax.axis_index("s")
    b0 = (s * NC + c) * PB
    ci = _ci(16)

    pltpu.sync_copy(ids.at[pl.ds(b0, PB)], ids_v)

    @pl.loop(0, nf * 8)
    def _(m):
        pv = m * 16 + ci
        f = pv >> 7
        bl = pv & (PB - 1)
        plsc.store_scatter(idx_v, [pv], plsc.load_gather(ids_v, [bl, f]))
        plsc.store_scatter(dst_v, [jnp.full((16,), m // 8, jnp.int32),
                                   (m % 8) * 16 + ci],
                           (b0 + bl) * nf + f)

    cps = [pltpu.async_copy(tab.at[f].at[idx_v.at[pl.ds(f * CH, CH)]],
                            rows_v.at[pl.ds(f * CH, CH)], sem)
           for f in range(nf)]
    for cp in cps:
        cp.wait()
    for f in range(nf):
        pltpu.sync_copy(rows_v.at[pl.ds(f * CH, CH)], out.at[dst_v.at[f]])


def _dyn_body(dtab, dids, dlens, zeros, out,
              idx_v, seg_v, dst_v, rows_v, dids_v, lens_v, inv_v,
              shared, sem):
    c = lax.axis_index("c")
    s = lax.axis_index("s")
    wid = s * NC + c
    b0 = wid * PB
    ci = _ci(16)

    pltpu.sync_copy(dids.at[pl.ds(b0, PB)], dids_v)
    pltpu.sync_copy(dlens.at[pl.ds(b0, PB)], lens_v)
    pltpu.sync_copy(zeros, shared.at[pl.ds(s * ACC_STRIDE, ACC_STRIDE)])
    accbase = s * ACC_STRIDE

    for f in range(FD):
        for t in range(DF_ROWS // DCH):
            @pl.loop(0, DCH // 16)
            def _(m):
                qf = t * DCH + m * 16 + ci
                bl = qf // L
                ln = qf - bl * L
                pairl = bl * FD + f
                fv = jnp.full((16,), f, jnp.int32)
                idxv = plsc.load_gather(dids_v, [bl, fv, ln])
                plsc.store_scatter(idx_v, [m * 16 + ci], idxv)
                lenv = plsc.load_gather(lens_v, [bl, fv])
                segv = jnp.where(ln < lenv, pairl, PAIRS) + accbase
                plsc.store_scatter(seg_v, [jnp.full((16,), m // 8, jnp.int32),
                                           (m % 8) * 16 + ci], segv)

            cps = [pltpu.async_copy(dtab.at[f].at[idx_v.at[pl.ds(j * CH, CH)]],
                                    rows_v.at[pl.ds(j * CH, CH)], sem)
                   for j in range(DCH // CH)]
            for cp in cps:
                cp.wait()
            for j in range(DCH // CH):
                pltpu.sync_copy(rows_v.at[pl.ds(j * CH, CH)],
                                shared.at[seg_v.at[j]], add=True)

    for i in range(PAIRS // 16):
        pv = i * 16 + ci
        lf = plsc.load_gather(lens_v, [pv >> 2, pv & 3]).astype(jnp.float32)
        inv_v[pl.ds(i * 16, 16)] = 1.0 / jnp.maximum(lf, 1.0)

    pltpu.sync_copy(shared.at[pl.ds(accbase, PAIRS)],
                    rows_v.at[pl.ds(0, PAIRS)])

    @pl.loop(0, PAIRS)
    def _(p):
        rowi = jnp.full((16,), p, dtype=jnp.int32)
        invs = plsc.load_gather(inv_v, [rowi])
        for h in range(2):
            col = ci + h * 16
            v = plsc.load_gather(rows_v, [rowi, col])
            plsc.store_scatter(rows_v, [rowi, col], v * invs)

    pltpu.sync_copy(rows_v.at[pl.ds(0, PAIRS)],
                    out.at[pl.ds(b0 * FD, PAIRS)])


def _mk_lookup(nf):
    return pl.kernel(
        functools.partial(_lookup_body, nf),
        out_type=jax.ShapeDtypeStruct((B * nf, D), jnp.float32),
        mesh=_mesh,
        scratch_types=[
            pltpu.VMEM((nf * CH,), jnp.int32),
            pltpu.VMEM((nf, CH), jnp.int32),
            pltpu.VMEM((nf * CH, D), jnp.float32),
            pltpu.VMEM((PB, nf), jnp.int32),
            pltpu.SemaphoreType.DMA,
        ],
        compiler_params=_cparams,
    )


_static_call = _mk_lookup(FS)
_ad_call = _mk_lookup(FA)

_dyn_call = pl.kernel(
    _dyn_body,
    out_type=jax.ShapeDtypeStruct((B * FD, D), jnp.float32),
    mesh=_mesh,
    scratch_types=[
        pltpu.VMEM((DCH,), jnp.int32),
        pltpu.VMEM((8, CH), jnp.int32),
        pltpu.VMEM((4, CH), jnp.int32),
        pltpu.VMEM((DCH, D), jnp.float32),
        pltpu.VMEM((PB, FD, L), jnp.int32),
        pltpu.VMEM((PB, FD), jnp.int32),
        pltpu.VMEM((PAIRS,), jnp.float32),
        pltpu.VMEM_SHARED((NS * ACC_STRIDE, D), jnp.float32),
        pltpu.SemaphoreType.DMA,
    ],
    compiler_params=_cparams,
)


def kernel(static_ids, ad_ids, dynamic_ids, dynamic_lengths,
           static_tables, ad_tables, dynamic_tables):
    zeros = jnp.zeros((ACC_STRIDE, D), jnp.float32)
    aout = _ad_call(ad_tables, ad_ids)
    dout = _dyn_call(dynamic_tables, dynamic_ids, dynamic_lengths, zeros)
    sout = _static_call(static_tables, static_ids)
    out1 = jnp.concatenate(
        [sout.reshape(B, FS, D), dout.reshape(B, FD, D)], axis=1)
    return (out1, aout.reshape(B, FA, D))
